# Initial kernel scaffold; baseline (speedup 1.0000x reference)
import jax
import jax.numpy as jnp
from jax.experimental import pallas as pl


def kernel(x, edge_index, W_att, W_data, W_out):
    raise NotImplementedError("write your pallas kernel here")


# speedup vs baseline: 6.4646x; 6.4646x over previous
import functools

import jax
import jax.numpy as jnp
from jax import lax
from jax.experimental import pallas as pl
from jax.experimental.pallas import tpu as pltpu
from jax.experimental.pallas import tpu_sc as plsc

EPS = 1e-7
N = 10000
E = 320000
D = 128
NC = 2
NS = 16
NW = NC * NS
PER_W = E // NW
BLK = 80
NBLK = PER_W // BLK
GRP = BLK // 16
ZONE = N // NS
BN = 1000


def _arcosh(z):
    z = jnp.clip(z, 1.0 + 1e-7)
    return jnp.log(z + jnp.sqrt((z - 1.0) * (z + 1.0)))


def _logmap0(xb, m0):
    ym = jnp.where(m0, 0.0, xb)
    n = jnp.clip(jnp.sqrt(jnp.sum(ym * ym, axis=-1, keepdims=True)), EPS)
    return _arcosh(xb[:, 0:1]) * ym / n


def _expmap0(mu, m0):
    ym = jnp.where(m0, 0.0, mu)
    n = jnp.clip(jnp.sqrt(jnp.sum(ym * ym, axis=-1, keepdims=True)), EPS)
    en = jnp.exp(n)
    yy = (0.5 * (en - 1.0 / en)) * ym / n
    x0 = jnp.sqrt(1.0 + jnp.sum(yy * yy, axis=-1, keepdims=True))
    return jnp.where(m0, x0, yy)


def _pre_body(x_ref, wa_ref, wd_ref, att_ref, p_ref):
    xb = x_ref[...]
    m0 = lax.broadcasted_iota(jnp.int32, (BN, D), 1) == 0
    u = _logmap0(xb, m0)
    att = _expmap0(jnp.dot(u, wa_ref[...], preferred_element_type=jnp.float32), m0)
    red = _expmap0(jnp.dot(u, wd_ref[...], preferred_element_type=jnp.float32), m0)
    att_ref[...] = att
    v = jnp.where(m0, 0.0, red / (red[:, 0:1] + 1.0))
    lam = 2.0 / jnp.clip(1.0 - jnp.sum(v * v, axis=-1, keepdims=True), EPS)
    p_ref[...] = jnp.where(m0, lam - 1.0, lam * v)


def _pre(x, wa_t, wd_t):
    return pl.pallas_call(
        _pre_body,
        grid=(N // BN,),
        in_specs=[
            pl.BlockSpec((BN, D), lambda i: (i, 0)),
            pl.BlockSpec((D, D), lambda i: (0, 0)),
            pl.BlockSpec((D, D), lambda i: (0, 0)),
        ],
        out_specs=[
            pl.BlockSpec((BN, D), lambda i: (i, 0)),
            pl.BlockSpec((BN, D), lambda i: (i, 0)),
        ],
        out_shape=[
            jax.ShapeDtypeStruct((N, D), jnp.float32),
            jax.ShapeDtypeStruct((N, D), jnp.float32),
        ],
    )(x, wa_t, wd_t)


_MESH = plsc.VectorSubcoreMesh(core_axis_name="c", subcore_axis_name="s")


@functools.partial(
    pl.kernel,
    out_type=jax.ShapeDtypeStruct((E,), jnp.float32),
    mesh=_MESH,
    compiler_params=pltpu.CompilerParams(needs_layout_passes=False),
    scratch_types=[
        pltpu.VMEM((2, BLK), jnp.int32),
        pltpu.VMEM((BLK, D), jnp.float32),
        pltpu.VMEM((BLK, D), jnp.float32),
        pltpu.VMEM((BLK,), jnp.float32),
        pltpu.SemaphoreType.DMA,
        pltpu.SemaphoreType.DMA,
    ],
)
def _sc_dot(att_hbm, eidx_hbm, prod_hbm,
            ebuf, rows_s, rows_d, prodv, sem1, sem2):
    cid = lax.axis_index("c")
    sid = lax.axis_index("s")
    wid = sid * NC + cid
    base_w = wid * PER_W

    lane = lax.iota(jnp.int32, 16)
    msk = jnp.where(lane == 0, -1.0, 1.0).astype(jnp.float32)

    def blk_body(i, _):
        pltpu.sync_copy(eidx_hbm.at[wid, i], ebuf)
        cp1 = pltpu.async_copy(att_hbm.at[ebuf.at[0]], rows_s, sem1)
        cp2 = pltpu.async_copy(att_hbm.at[ebuf.at[1]], rows_d, sem2)
        cp1.wait()
        cp2.wait()

        def grp_body(g, _):
            res = jnp.zeros((16,), jnp.float32)
            for e16 in range(16):
                e = g * 16 + e16
                acc = rows_s[e, pl.ds(0, 16)] * rows_d[e, pl.ds(0, 16)] * msk
                for k in range(1, D // 16):
                    acc = acc + (rows_s[e, pl.ds(k * 16, 16)]
                                 * rows_d[e, pl.ds(k * 16, 16)])
                res = jnp.where(lane == e16, jnp.sum(acc), res)
            prodv[pl.ds(g * 16, 16)] = res
            return 0

        lax.fori_loop(0, GRP, grp_body, 0)
        pltpu.sync_copy(prodv, prod_hbm.at[pl.ds(base_w + i * BLK, BLK)])
        return 0

    lax.fori_loop(0, NBLK, blk_body, 0)


def _coef_body(prod_ref, coef_ref):
    z = jnp.clip(-prod_ref[...], 1.0 + 1e-7)
    a = _arcosh(z)
    coef_ref[...] = -(a * a)


def _coef(prod2d):
    return pl.pallas_call(
        _coef_body,
        out_shape=jax.ShapeDtypeStruct(prod2d.shape, jnp.float32),
    )(prod2d)


@functools.partial(
    pl.kernel,
    out_type=jax.ShapeDtypeStruct((NC, N, D), jnp.float32),
    mesh=_MESH,
    scratch_types=[
        pltpu.VMEM((2, BLK), jnp.int32),
        pltpu.VMEM((BLK,), jnp.float32),
        pltpu.VMEM((BLK, D), jnp.float32),
        pltpu.VMEM_SHARED((N, D), jnp.float32),
        pltpu.SemaphoreType.DMA,
    ],
)
def _sc_agg(p_hbm, eidx_hbm, coef_hbm, out_hbm,
            ebuf, coefb, rows, acc, sem):
    cid = lax.axis_index("c")
    sid = lax.axis_index("s")
    wid = sid * NC + cid

    zero16 = jnp.zeros((16,), jnp.float32)

    def zrow(j, _):
        rj = rows.at[j]
        for k in range(D // 16):
            rj[pl.ds(k * 16, 16)] = zero16
        return 0

    lax.fori_loop(0, BLK, zrow, 0)
    nchunk = N // BLK

    def zchunk(t, _):
        j = sid + t * NS

        @pl.when(j < nchunk)
        def _():
            pltpu.sync_copy(rows, acc.at[pl.ds(j * BLK, BLK)])

        return 0

    lax.fori_loop(0, (nchunk + NS - 1) // NS, zchunk, 0)
    plsc.subcore_barrier()

    def blk_body(i, _):
        pltpu.sync_copy(eidx_hbm.at[wid, i], ebuf)
        pltpu.sync_copy(coef_hbm.at[wid, i], coefb)
        pltpu.async_copy(p_hbm.at[ebuf.at[0]], rows, sem).wait()

        def scale(g, _):
            cvec = coefb[pl.ds(g * 16, 16)]
            for e16 in range(16):
                re = rows.at[g * 16 + e16]
                bc = lax.broadcast(cvec[e16], (16,))
                for k in range(D // 16):
                    re[pl.ds(k * 16, 16)] = re[pl.ds(k * 16, 16)] * bc
            return 0

        lax.fori_loop(0, GRP, scale, 0)
        pltpu.sync_copy(rows, acc.at[ebuf.at[1]], add=True)
        return 0

    lax.fori_loop(0, NBLK, blk_body, 0)
    plsc.subcore_barrier()

    def ochunk(t, _):
        j = sid + t * NS

        @pl.when(j < nchunk)
        def _():
            pltpu.sync_copy(acc.at[pl.ds(j * BLK, BLK)],
                            out_hbm.at[cid, pl.ds(j * BLK, BLK)])

        return 0

    lax.fori_loop(0, (nchunk + NS - 1) // NS, ochunk, 0)


def _post_body(a0_ref, a1_ref, wo_ref, out_ref):
    s = a0_ref[...] + a1_ref[...]
    m0 = lax.broadcasted_iota(jnp.int32, (BN, D), 1) == 0
    den = s[:, 0:1]
    den = jnp.where(jnp.abs(den) < EPS, EPS, den)
    m = jnp.where(m0, 0.0, s) / den
    mn = jnp.sqrt(jnp.sum(m * m, axis=-1, keepdims=True))
    m = m * (jnp.clip(mn, None, 1.0 - 1e-5) / jnp.clip(mn, EPS))
    n = jnp.clip(jnp.sqrt(jnp.sum(m * m, axis=-1, keepdims=True)), EPS)
    nc = jnp.clip(n, EPS, 1.0 - 1e-5)
    half = nc / (1.0 + jnp.sqrt((1.0 - nc) * (1.0 + nc)))
    mid = half * m / n
    n2 = jnp.sum(mid * mid, axis=-1, keepdims=True)
    dnm = jnp.clip(1.0 - n2, EPS)
    h = jnp.where(m0, (1.0 + n2) / dnm, 2.0 * mid / dnm)
    u = _logmap0(h, m0)
    out_ref[...] = _expmap0(
        jnp.dot(u, wo_ref[...], preferred_element_type=jnp.float32), m0)


def _post(a0, a1, wo_t):
    return pl.pallas_call(
        _post_body,
        grid=(N // BN,),
        in_specs=[
            pl.BlockSpec((BN, D), lambda i: (i, 0)),
            pl.BlockSpec((BN, D), lambda i: (i, 0)),
            pl.BlockSpec((D, D), lambda i: (0, 0)),
        ],
        out_specs=pl.BlockSpec((BN, D), lambda i: (i, 0)),
        out_shape=jax.ShapeDtypeStruct((N, D), jnp.float32),
    )(a0, a1, wo_t)


def kernel(x, edge_index, W_att, W_data, W_out):
    att, p = _pre(x, W_att.T, W_data.T)
    eidx = jnp.stack(
        [edge_index[0].reshape(NW, NBLK, BLK),
         edge_index[1].reshape(NW, NBLK, BLK)], axis=2)
    prod = _sc_dot(att, eidx)
    coef = _coef(prod.reshape(E // D, D)).reshape(NW, NBLK, BLK)
    part = _sc_agg(p, eidx, coef)
    return _post(part[0], part[1], W_out.T)


# speedup vs baseline: 16.5926x; 2.5667x over previous
import functools

import jax
import jax.numpy as jnp
from jax import lax
from jax.experimental import pallas as pl
from jax.experimental.pallas import tpu as pltpu
from jax.experimental.pallas import tpu_sc as plsc

EPS = 1e-7
N = 10000
E = 320000
D = 128
NC = 2
NS = 16
NW = NC * NS
BLK = 64
GRP = BLK // 16
NBLK = 160
NG = NBLK // 8
PER_W = NBLK * BLK
ZBLK = 40
BN = 1000


def _arcosh(z):
    z = jnp.clip(z, 1.0 + 1e-7)
    return jnp.log(z + jnp.sqrt((z - 1.0) * (z + 1.0)))


def _logmap0(xb, m0):
    ym = jnp.where(m0, 0.0, xb)
    n = jnp.clip(jnp.sqrt(jnp.sum(ym * ym, axis=-1, keepdims=True)), EPS)
    return _arcosh(xb[:, 0:1]) * ym / n


def _expmap0(mu, m0):
    ym = jnp.where(m0, 0.0, mu)
    n = jnp.clip(jnp.sqrt(jnp.sum(ym * ym, axis=-1, keepdims=True)), EPS)
    en = jnp.exp(n)
    yy = (0.5 * (en - 1.0 / en)) * ym / n
    x0 = jnp.sqrt(1.0 + jnp.sum(yy * yy, axis=-1, keepdims=True))
    return jnp.where(m0, x0, yy)


def _pre_body(x_ref, wa_ref, wd_ref, att_ref, p_ref):
    xb = x_ref[...]
    m0 = lax.broadcasted_iota(jnp.int32, (BN, D), 1) == 0
    u = _logmap0(xb, m0)
    att = _expmap0(jnp.dot(u, wa_ref[...], preferred_element_type=jnp.float32), m0)
    red = _expmap0(jnp.dot(u, wd_ref[...], preferred_element_type=jnp.float32), m0)
    att_ref[...] = att
    v = jnp.where(m0, 0.0, red / (red[:, 0:1] + 1.0))
    lam = 2.0 / jnp.clip(1.0 - jnp.sum(v * v, axis=-1, keepdims=True), EPS)
    p_ref[...] = jnp.where(m0, lam - 1.0, lam * v)


def _pre(x, wa_t, wd_t):
    return pl.pallas_call(
        _pre_body,
        grid=(N // BN,),
        in_specs=[
            pl.BlockSpec((BN, D), lambda i: (i, 0)),
            pl.BlockSpec((D, D), lambda i: (0, 0)),
            pl.BlockSpec((D, D), lambda i: (0, 0)),
        ],
        out_specs=[
            pl.BlockSpec((BN, D), lambda i: (i, 0)),
            pl.BlockSpec((BN, D), lambda i: (i, 0)),
        ],
        out_shape=[
            jax.ShapeDtypeStruct((N, D), jnp.float32),
            jax.ShapeDtypeStruct((N, D), jnp.float32),
        ],
    )(x, wa_t, wd_t)


_MESH = plsc.VectorSubcoreMesh(core_axis_name="c", subcore_axis_name="s")


def _sc_ln(z):
    bits = lax.bitcast_convert_type(z, jnp.int32)
    e = (lax.shift_right_logical(bits, 23) & 255) - 127
    m = lax.bitcast_convert_type((bits & 0x7FFFFF) | 0x3F800000, jnp.float32)
    big = m > 1.4142135
    m = jnp.where(big, m * 0.5, m)
    e = jnp.where(big, e + 1, e)
    u = (m - 1.0) / (m + 1.0)
    u2 = u * u
    p = 2.0 * u * (1.0 + u2 * (1.0 / 3 + u2 * (1.0 / 5 + u2 * (1.0 / 7 + u2 / 9))))
    return e.astype(jnp.float32) * 0.69314718 + p


def _sc_coef(prod):
    z = jnp.maximum(-prod, 1.0 + 1e-7)
    w = (z - 1.0) * (z + 1.0)
    s = jnp.exp(0.5 * _sc_ln(w))
    a = _sc_ln(z + s)
    return -(a * a)


@functools.partial(
    pl.kernel,
    out_type=jax.ShapeDtypeStruct((NC, N, D), jnp.float32),
    mesh=_MESH,
    compiler_params=pltpu.CompilerParams(needs_layout_passes=False),
    scratch_types=[
        pltpu.VMEM((8, D), jnp.int32),
        pltpu.VMEM((BLK,), jnp.int32),
        pltpu.VMEM((BLK,), jnp.float32),
        pltpu.VMEM((BLK, D), jnp.float32),
        pltpu.VMEM((BLK, D), jnp.float32),
        pltpu.VMEM((BLK, D), jnp.float32),
        pltpu.VMEM((BLK, D), jnp.float32),
        pltpu.VMEM((BLK, D), jnp.float32),
        pltpu.VMEM((BLK, D), jnp.float32),
        pltpu.VMEM_SHARED((N, D), jnp.float32),
        pltpu.SemaphoreType.DMA,
        pltpu.SemaphoreType.DMA,
        pltpu.SemaphoreType.DMA,
        pltpu.SemaphoreType.DMA,
        pltpu.SemaphoreType.DMA,
        pltpu.SemaphoreType.DMA,
        pltpu.SemaphoreType.DMA,
        pltpu.SemaphoreType.DMA,
    ],
)
def _sc_edge(att_hbm, p_hbm, e4_hbm, out_hbm,
             ebuf, dstb, cbuf, sA, dA, pA, sB, dB, pB, acc,
             semSA, semDA, semPA, semCA, semSB, semDB, semPB, semCB):
    cid = lax.axis_index("c")
    sid = lax.axis_index("s")
    wid = sid * NC + cid
    ebase = wid * PER_W

    lane = lax.iota(jnp.int32, 16)
    msk = jnp.where(lane == 0, -1.0, 1.0).astype(jnp.float32)
    zero16 = jnp.zeros((16,), jnp.float32)

    def zrow(j, _):
        rj = pA.at[j]
        for k in range(D // 16):
            rj[pl.ds(k * 16, 16)] = zero16
        return 0

    lax.fori_loop(0, ZBLK, zrow, 0)
    nchunk = N // ZBLK

    def zchunk(t, _):
        j = sid + t * NS

        @pl.when(j < nchunk)
        def _():
            pltpu.sync_copy(pA.at[pl.ds(0, ZBLK)], acc.at[pl.ds(j * ZBLK, ZBLK)])

        return 0

    lax.fori_loop(0, (nchunk + NS - 1) // NS, zchunk, 0)
    plsc.subcore_barrier()

    def _valid(t):
        return ebase + t * BLK < E

    def _issue(t, s_buf, d_buf, p_buf, sem_s, sem_d, sem_p):
        r = t & 7
        src_idx = ebuf.at[r, pl.ds(0, BLK)]
        dst_idx = ebuf.at[r, pl.ds(BLK, BLK)]
        pltpu.async_copy(att_hbm.at[src_idx], s_buf, sem_s)
        pltpu.async_copy(att_hbm.at[dst_idx], d_buf, sem_d)
        pltpu.async_copy(p_hbm.at[src_idx], p_buf, sem_p)

    def _half(t, cur, nxt):
        (cs, cd, cp, csemS, csemD, csemP, csemC) = cur
        (ns_, nd_, np_, nsemS, nsemD, nsemP, nsemC) = nxt
        r = t & 7
        validt = _valid(t)

        @pl.when(validt)
        def _():
            er = ebuf.at[r]
            for k in range(GRP):
                dstb[pl.ds(k * 16, 16)] = er[pl.ds(BLK + k * 16, 16)]

        @pl.when(((t & 7) == 7) & (t + 1 < NBLK))
        def _():
            pltpu.sync_copy(e4_hbm.at[wid, (t + 1) >> 3], ebuf)

        @pl.when(_valid(t + 1) & (t + 1 < NBLK))
        def _():
            rn = (t + 1) & 7
            pltpu.async_copy(att_hbm.at[ebuf.at[rn, pl.ds(0, BLK)]], ns_, nsemS)
            pltpu.async_copy(att_hbm.at[ebuf.at[rn, pl.ds(BLK, BLK)]], nd_, nsemD)

        @pl.when(validt)
        def _():
            pltpu.make_async_copy(att_hbm.at[pl.ds(0, BLK)], cs, csemS).wait()
            pltpu.make_async_copy(att_hbm.at[pl.ds(0, BLK)], cd, csemD).wait()

            def dot_grp(g, _):
                def dot_body(e16, res):
                    e = g * 16 + e16
                    acc_d = cs[e, pl.ds(0, 16)] * cd[e, pl.ds(0, 16)] * msk
                    for k in range(1, D // 16):
                        acc_d = acc_d + (cs[e, pl.ds(k * 16, 16)]
                                         * cd[e, pl.ds(k * 16, 16)])
                    return jnp.where(lane == e16, jnp.sum(acc_d), res)

                res = lax.fori_loop(0, 16, dot_body, zero16, unroll=4)
                cbuf[pl.ds(g * 16, 16)] = _sc_coef(res)
                return 0

            lax.fori_loop(0, GRP, dot_grp, 0)
            pltpu.make_async_copy(p_hbm.at[pl.ds(0, BLK)], cp, csemP).wait()

            def scale_grp(g, _):
                cvec = cbuf[pl.ds(g * 16, 16)]
                for e16 in range(16):
                    re = cp.at[g * 16 + e16]
                    bc = lax.broadcast(cvec[e16], (16,))
                    for k in range(D // 16):
                        re[pl.ds(k * 16, 16)] = re[pl.ds(k * 16, 16)] * bc
                didx = dstb[pl.ds(g * 16, 16)]
                pltpu.async_copy(cp.at[pl.ds(g * 16, 16)], acc.at[didx],
                                 csemC, add=True)
                return 0

            lax.fori_loop(0, GRP, scale_grp, 0)

        @pl.when((t >= 1) & _valid(t - 1))
        def _():
            pltpu.make_async_copy(p_hbm.at[pl.ds(0, BLK)], np_, nsemC).wait()

        @pl.when(_valid(t + 1) & (t + 1 < NBLK))
        def _():
            rn = (t + 1) & 7
            pltpu.async_copy(p_hbm.at[ebuf.at[rn, pl.ds(0, BLK)]], np_, nsemP)

        return None

    bufA = (sA, dA, pA, semSA, semDA, semPA, semCA)
    bufB = (sB, dB, pB, semSB, semDB, semPB, semCB)

    pltpu.sync_copy(e4_hbm.at[wid, 0], ebuf)
    _issue(0, sA, dA, pA, semSA, semDA, semPA)

    def pair_body(it, _):
        _half(2 * it, bufA, bufB)
        _half(2 * it + 1, bufB, bufA)
        return 0

    lax.fori_loop(0, NBLK // 2, pair_body, 0)

    @pl.when(_valid(NBLK - 1))
    def _():
        pltpu.make_async_copy(p_hbm.at[pl.ds(0, BLK)], pB, semCB).wait()

    plsc.subcore_barrier()

    def ochunk(t, _):
        j = sid + t * NS

        @pl.when(j < nchunk)
        def _():
            pltpu.sync_copy(acc.at[pl.ds(j * ZBLK, ZBLK)],
                            out_hbm.at[cid, pl.ds(j * ZBLK, ZBLK)])

        return 0

    lax.fori_loop(0, (nchunk + NS - 1) // NS, ochunk, 0)


def _post_body(a0_ref, a1_ref, wo_ref, out_ref):
    s = a0_ref[...] + a1_ref[...]
    m0 = lax.broadcasted_iota(jnp.int32, (BN, D), 1) == 0
    den = s[:, 0:1]
    den = jnp.where(jnp.abs(den) < EPS, EPS, den)
    m = jnp.where(m0, 0.0, s) / den
    mn = jnp.sqrt(jnp.sum(m * m, axis=-1, keepdims=True))
    m = m * (jnp.clip(mn, None, 1.0 - 1e-5) / jnp.clip(mn, EPS))
    n = jnp.clip(jnp.sqrt(jnp.sum(m * m, axis=-1, keepdims=True)), EPS)
    nc = jnp.clip(n, EPS, 1.0 - 1e-5)
    half = nc / (1.0 + jnp.sqrt((1.0 - nc) * (1.0 + nc)))
    mid = half * m / n
    n2 = jnp.sum(mid * mid, axis=-1, keepdims=True)
    dnm = jnp.clip(1.0 - n2, EPS)
    h = jnp.where(m0, (1.0 + n2) / dnm, 2.0 * mid / dnm)
    u = _logmap0(h, m0)
    out_ref[...] = _expmap0(
        jnp.dot(u, wo_ref[...], preferred_element_type=jnp.float32), m0)


def _post(a0, a1, wo_t):
    return pl.pallas_call(
        _post_body,
        grid=(N // BN,),
        in_specs=[
            pl.BlockSpec((BN, D), lambda i: (i, 0)),
            pl.BlockSpec((BN, D), lambda i: (i, 0)),
            pl.BlockSpec((D, D), lambda i: (0, 0)),
        ],
        out_specs=pl.BlockSpec((BN, D), lambda i: (i, 0)),
        out_shape=jax.ShapeDtypeStruct((N, D), jnp.float32),
    )(a0, a1, wo_t)


def kernel(x, edge_index, W_att, W_data, W_out):
    att, p = _pre(x, W_att.T, W_data.T)
    pad = NW * PER_W - E
    src_p = jnp.concatenate([edge_index[0], jnp.zeros((pad,), jnp.int32)])
    dst_p = jnp.concatenate([edge_index[1], jnp.zeros((pad,), jnp.int32)])
    e4 = jnp.concatenate(
        [src_p.reshape(NW, NG, 8, BLK), dst_p.reshape(NW, NG, 8, BLK)],
        axis=3)
    part = _sc_edge(att, p, e4)
    return _post(part[0], part[1], W_out.T)
